# baseline (speedup 1.0000x reference)
import functools

import jax
import jax.numpy as jnp
from jax import lax
from jax.experimental import pallas as pl
from jax.experimental.pallas import tpu as pltpu
from jax.experimental.pallas import tpu_sc as plsc

N = 1_000_000
D = 16
B = 16384

NC = 2
NS = 16
NW = NC * NS
BW = B // NW
ROWS = 3 * BW
CHUNK = 128
NCHUNK = ROWS // CHUNK
TCHUNK = 4
TROWS = ROWS // TCHUNK
TTRIP = BW // TCHUNK

LN2 = 0.6931471805599453
_P = (
    5.06533310e-06,
    1.44239548e+00,
    -7.16986875e-01,
    4.53856241e-01,
    -2.72353158e-01,
    1.17905183e-01,
    -2.48256066e-02,
)

_mesh = plsc.VectorSubcoreMesh(core_axis_name="c", subcore_axis_name="s")


def _softplus16(x):
    y = 1.0 + jnp.exp(x)
    bits = lax.bitcast_convert_type(y, jnp.int32)
    ex = jnp.right_shift(bits, 23) - 127
    m = lax.bitcast_convert_type(
        jnp.bitwise_or(jnp.bitwise_and(bits, 0x007FFFFF), 0x3F800000),
        jnp.float32,
    )
    t = m - 1.0
    p = jnp.float32(_P[6])
    for k in (5, 4, 3, 2, 1, 0):
        p = p * t + jnp.float32(_P[k])
    log2y = ex.astype(jnp.float32) + p
    return log2y * jnp.float32(LN2)


@functools.partial(
    pl.kernel,
    mesh=_mesh,
    out_type=jax.ShapeDtypeStruct((B,), jnp.float32),
    compiler_params=pltpu.CompilerParams(
        needs_layout_passes=False, use_tc_tiling_on_sc=True
    ),
    scratch_types=[
        pltpu.VMEM((NCHUNK, CHUNK), jnp.int32),
        pltpu.VMEM((NCHUNK, CHUNK), jnp.int32),
        pltpu.VMEM((TROWS, CHUNK), jnp.float32),
        pltpu.VMEM((BW,), jnp.float32),
        pltpu.SemaphoreType.DMA,
    ],
)
def _ste_sc(idx_hbm, table8_hbm, out_hbm, idx_v, q_v, buf_v, out_v, sem):
    wid = lax.axis_index("s") * NC + lax.axis_index("c")
    base = wid * BW

    pltpu.sync_copy(idx_hbm.at[wid], idx_v)

    for k in range(NCHUNK):
        for jv in range(CHUNK // 16):
            sl = pl.ds(jv * 16, 16)
            q_v[k, sl] = jnp.right_shift(idx_v[k, sl], 3)

    lane = lax.iota(jnp.int32, 16)

    for t in range(TCHUNK):
        copies = [
            pltpu.async_copy(
                table8_hbm.at[q_v.at[3 * t + s]],
                buf_v.at[pl.ds(s * CHUNK, CHUNK)],
                sem,
            )
            for s in range(3)
        ]
        for cp in copies:
            cp.wait()

        def group(g, carry, t=t):
            acc = jnp.zeros((16,), jnp.float32)
            p0 = t * TROWS + g * 48
            rof = [None] * 3
            for s in range(3):
                p = p0 + 3 * lane + s
                rv = plsc.load_gather(
                    idx_v, [jnp.right_shift(p, 7), jnp.bitwise_and(p, 127)]
                )
                rof[s] = jnp.left_shift(jnp.bitwise_and(rv, 7), 4)
            for j in range(16):
                il = g * 16 + j
                vecs = [None] * 3
                for s in range(3):
                    off = lax.squeeze(lax.slice(rof[s], (j,), (j + 1,)), (0,))
                    vecs[s] = buf_v[3 * il + s, pl.ds(off, 16)]
                dw = vecs[0] - vecs[1]
                dl = vecs[0] - vecs[2]
                v = dw * dw - dl * dl
                acc = jnp.where(lane == j, jnp.sum(v), acc)
            out_v[pl.ds(t * TTRIP + g * 16, 16)] = _softplus16(acc)
            return carry

        lax.fori_loop(0, TTRIP // 16, group, 0)

    pltpu.sync_copy(out_v, out_hbm.at[pl.ds(base, BW)])


def kernel(h_w_l, embedding):
    idx = h_w_l.reshape(NW, NCHUNK, CHUNK)
    return _ste_sc(idx, embedding.reshape(N // 8, 8 * D))


# speedup vs baseline: 1.3050x; 1.3050x over previous
import functools

import jax
import jax.numpy as jnp
from jax import lax
from jax.experimental import pallas as pl
from jax.experimental.pallas import tpu as pltpu
from jax.experimental.pallas import tpu_sc as plsc

N = 1_000_000
D = 16
B = 16384

NC = 2
NS = 16
NW = NC * NS
BW = B // NW
ROWS = 3 * BW
NIDX = ROWS // 128
RND = 96
NROUND = ROWS // RND
RTRIP = RND // 3

LN2 = 0.6931471805599453
_P = (
    5.06533310e-06,
    1.44239548e+00,
    -7.16986875e-01,
    4.53856241e-01,
    -2.72353158e-01,
    1.17905183e-01,
    -2.48256066e-02,
)

_mesh = plsc.VectorSubcoreMesh(core_axis_name="c", subcore_axis_name="s")


def _softplus16(x):
    y = 1.0 + jnp.exp(x)
    bits = lax.bitcast_convert_type(y, jnp.int32)
    ex = jnp.right_shift(bits, 23) - 127
    m = lax.bitcast_convert_type(
        jnp.bitwise_or(jnp.bitwise_and(bits, 0x007FFFFF), 0x3F800000),
        jnp.float32,
    )
    t = m - 1.0
    p = jnp.float32(_P[6])
    for k in (5, 4, 3, 2, 1, 0):
        p = p * t + jnp.float32(_P[k])
    log2y = ex.astype(jnp.float32) + p
    return log2y * jnp.float32(LN2)


@functools.partial(
    pl.kernel,
    mesh=_mesh,
    out_type=jax.ShapeDtypeStruct((B,), jnp.float32),
    compiler_params=pltpu.CompilerParams(
        needs_layout_passes=False, use_tc_tiling_on_sc=True
    ),
    scratch_types=[
        pltpu.VMEM((NIDX, 128), jnp.int32),
        pltpu.VMEM((RND, 8, D), jnp.float32),
        pltpu.VMEM((BW,), jnp.float32),
        pltpu.SemaphoreType.DMA,
    ],
)
def _ste_sc(idx_hbm, table_hbm, out_hbm, idx_v, buf_v, out_v, sem):
    wid = lax.axis_index("s") * NC + lax.axis_index("c")
    base = wid * BW
    t8 = table_hbm.reshape(N // 8, 8, D)

    pltpu.sync_copy(idx_hbm.at[wid], idx_v)

    lane = lax.iota(jnp.int32, 16)

    def round_body(c, carry):
        p0 = c * RND
        for m in range(RND // 16):
            pm = p0 + 16 * m
            rvec = idx_v[
                jnp.right_shift(pm, 7), pl.ds(jnp.bitwise_and(pm, 127), 16)
            ]
            for j in range(16):
                q8 = pl.multiple_of(jnp.bitwise_and(rvec[j], ~7), 8)
                pltpu.async_copy(
                    table_hbm.at[pl.ds(q8, 8), :],
                    buf_v.at[16 * m + j],
                    sem,
                )
        pltpu.make_async_copy(t8.at[pl.ds(0, RND)], buf_v, sem).wait()

        for gl in range(RTRIP // 16):
            acc = jnp.zeros((16,), jnp.float32)
            rv = [None] * 3
            for m in range(3):
                pm = p0 + 48 * gl + 16 * m
                rv[m] = idx_v[
                    jnp.right_shift(pm, 7), pl.ds(jnp.bitwise_and(pm, 127), 16)
                ]
            for j in range(16):
                il = gl * 16 + j
                vecs = [None] * 3
                for s in range(3):
                    q = 3 * j + s
                    r = rv[q // 16][q % 16]
                    vecs[s] = buf_v[3 * il + s, jnp.bitwise_and(r, 7), :]
                dw = vecs[0] - vecs[1]
                dl = vecs[0] - vecs[2]
                v = dw * dw - dl * dl
                acc = jnp.where(lane == j, jnp.sum(v), acc)
            out_v[pl.ds(c * RTRIP + gl * 16, 16)] = _softplus16(acc)
        return carry

    lax.fori_loop(0, NROUND, round_body, 0)

    pltpu.sync_copy(out_v, out_hbm.at[pl.ds(base, BW)])


def kernel(h_w_l, embedding):
    idx = h_w_l.reshape(NW, NIDX, 128)
    return _ste_sc(idx, embedding)


# speedup vs baseline: 1.3170x; 1.0092x over previous
import functools

import jax
import jax.numpy as jnp
from jax import lax
from jax.experimental import pallas as pl
from jax.experimental.pallas import tpu as pltpu
from jax.experimental.pallas import tpu_sc as plsc

N = 1_000_000
D = 16
B = 16384

NC = 2
NS = 16
NW = NC * NS
BW = B // NW
ROWS = 3 * BW
NIDX = ROWS // 128
RND = 48
NROUND = ROWS // RND
RTRIP = RND // 3

LN2 = 0.6931471805599453
_P = (
    5.06533310e-06,
    1.44239548e+00,
    -7.16986875e-01,
    4.53856241e-01,
    -2.72353158e-01,
    1.17905183e-01,
    -2.48256066e-02,
)

_mesh = plsc.VectorSubcoreMesh(core_axis_name="c", subcore_axis_name="s")


def _softplus16(x):
    y = 1.0 + jnp.exp(x)
    bits = lax.bitcast_convert_type(y, jnp.int32)
    ex = jnp.right_shift(bits, 23) - 127
    m = lax.bitcast_convert_type(
        jnp.bitwise_or(jnp.bitwise_and(bits, 0x007FFFFF), 0x3F800000),
        jnp.float32,
    )
    t = m - 1.0
    p = jnp.float32(_P[6])
    for k in (5, 4, 3, 2, 1, 0):
        p = p * t + jnp.float32(_P[k])
    log2y = ex.astype(jnp.float32) + p
    return log2y * jnp.float32(LN2)


@functools.partial(
    pl.kernel,
    mesh=_mesh,
    out_type=jax.ShapeDtypeStruct((B,), jnp.float32),
    compiler_params=pltpu.CompilerParams(
        needs_layout_passes=False, use_tc_tiling_on_sc=True
    ),
    scratch_types=[
        pltpu.VMEM((NIDX, 128), jnp.int32),
        pltpu.VMEM((RND, 8, D), jnp.float32),
        pltpu.VMEM((RND, 8, D), jnp.float32),
        pltpu.VMEM((BW,), jnp.float32),
        pltpu.SemaphoreType.DMA,
        pltpu.SemaphoreType.DMA,
    ],
)
def _ste_sc(idx_hbm, table_hbm, out_hbm, idx_v, buf_a, buf_b, out_v, sem_a, sem_b):
    wid = lax.axis_index("s") * NC + lax.axis_index("c")
    base = wid * BW
    t8 = table_hbm.reshape(N // 8, 8, D)

    pltpu.sync_copy(idx_hbm.at[wid], idx_v)

    lane = lax.iota(jnp.int32, 16)

    def fire(rnd, buf, sem):
        p0 = rnd * RND
        for m in range(RND // 16):
            pm = p0 + 16 * m
            rvec = idx_v[
                jnp.right_shift(pm, 7), pl.ds(jnp.bitwise_and(pm, 127), 16)
            ]
            for j in range(16):
                q8 = pl.multiple_of(jnp.bitwise_and(rvec[j], ~7), 8)
                pltpu.async_copy(
                    table_hbm.at[pl.ds(q8, 8), :],
                    buf.at[16 * m + j],
                    sem,
                )

    def drain(buf, sem):
        pltpu.make_async_copy(t8.at[pl.ds(0, RND)], buf, sem).wait()

    def compute(rnd, buf):
        p0 = rnd * RND
        acc = jnp.zeros((16,), jnp.float32)
        rv = [None] * 3
        for m in range(3):
            pm = p0 + 16 * m
            rv[m] = idx_v[
                jnp.right_shift(pm, 7), pl.ds(jnp.bitwise_and(pm, 127), 16)
            ]
        for j in range(16):
            vecs = [None] * 3
            for s in range(3):
                q = 3 * j + s
                r = rv[q // 16][q % 16]
                vecs[s] = buf[3 * j + s, jnp.bitwise_and(r, 7), :]
            dw = vecs[0] - vecs[1]
            dl = vecs[0] - vecs[2]
            v = dw * dw - dl * dl
            acc = jnp.where(lane == j, jnp.sum(v), acc)
        out_v[pl.ds(rnd * RTRIP, 16)] = _softplus16(acc)

    fire(0, buf_a, sem_a)

    def pair_body(c, carry):
        fire(2 * c + 1, buf_b, sem_b)
        drain(buf_a, sem_a)
        compute(2 * c, buf_a)

        @pl.when(c < NROUND // 2 - 1)
        def _():
            fire(2 * c + 2, buf_a, sem_a)

        drain(buf_b, sem_b)
        compute(2 * c + 1, buf_b)
        return carry

    lax.fori_loop(0, NROUND // 2, pair_body, 0)

    pltpu.sync_copy(out_v, out_hbm.at[pl.ds(base, BW)])


def kernel(h_w_l, embedding):
    idx = h_w_l.reshape(NW, NIDX, 128)
    return _ste_sc(idx, embedding)
